# baseline (speedup 1.0000x reference)
import functools

import jax
import jax.numpy as jnp
from jax import lax
from jax.experimental import pallas as pl
from jax.experimental.pallas import tpu as pltpu
from jax.experimental.pallas import tpu_sc as plsc

N = 10000
E = 320000
D_IN = 128
H1, H2, H3 = 128, 64, 32

NC = 2
NS = 16
NW = NC * NS
EPW = E // NW
NP = 10240
RPT = NP // NS
DEG_K = 80
DEG_NCH = EPW // DEG_K
DEGW = 16

_BN_EPS = 1e-5
_BR = 5000


_KCFG = {H1: 72, H2: 80, H3: 80}
_NBUFCFG = {H1: 3, H2: 3, H3: 3}


def _epwp(k, nbuf):
    n = -(-EPW // k)
    while n % nbuf != nbuf - 1:
        n += 1
    return n * k, n


def _make_propagate(D):
    k = _KCFG[D]
    nbuf = _NBUFCFG[D]
    epwp, nchp = _epwp(k, nbuf)
    assert nchp % nbuf == nbuf - 1
    mesh = plsc.VectorSubcoreMesh(core_axis_name="c", subcore_axis_name="s")

    @functools.partial(
        pl.kernel,
        out_type=jax.ShapeDtypeStruct((NC, NP, D), jnp.float32),
        mesh=mesh,
        scratch_types=(
            [pltpu.VMEM((nchp, k), jnp.int32),
             pltpu.VMEM((nchp, k), jnp.int32)]
            + [pltpu.VMEM((k, D), jnp.float32) for _ in range(nbuf)]
            + [pltpu.VMEM_SHARED((NP, D), jnp.float32)]
            + [pltpu.SemaphoreType.DMA for _ in range(nbuf)]
        ),
        compiler_params=pltpu.CompilerParams(use_tc_tiling_on_sc=False),
    )
    def prop(src_hbm, dst_hbm, y_hbm, zeros_hbm, out_hbm,
             src_v, dst_v, *bufs):
        rows = list(bufs[0:nbuf])
        acc_sh = bufs[nbuf]
        sems = list(bufs[nbuf + 1:2 * nbuf + 1])
        c = lax.axis_index("c")
        s = lax.axis_index("s")
        wid = s * NC + c
        pltpu.sync_copy(src_hbm.at[wid], src_v)
        pltpu.sync_copy(dst_hbm.at[wid], dst_v)
        pltpu.sync_copy(zeros_hbm, acc_sh.at[pl.ds(s * RPT, RPT)])
        plsc.subcore_barrier()

        for b in range(nbuf - 1):
            pltpu.async_copy(y_hbm.at[src_v.at[b]], rows[b], sems[b])

        def body(i, carry):
            for b in range(nbuf):
                j = nbuf * i + b
                kn = (b + nbuf - 1) % nbuf
                pltpu.async_copy(y_hbm.at[src_v.at[j + nbuf - 1]],
                                 rows[kn], sems[kn])
                pltpu.make_async_copy(y_hbm.at[src_v.at[j]], rows[b],
                                      sems[b]).wait()
                pltpu.sync_copy(rows[b], acc_sh.at[dst_v.at[j]], add=True)
            return carry

        lax.fori_loop(0, (nchp - (nbuf - 1)) // nbuf, body, 0)
        for t in range(nbuf - 1):
            j = nchp - (nbuf - 1) + t
            b = j % nbuf
            pltpu.make_async_copy(y_hbm.at[src_v.at[j]], rows[b],
                                  sems[b]).wait()
            pltpu.sync_copy(rows[b], acc_sh.at[dst_v.at[j]], add=True)
        plsc.subcore_barrier()
        pltpu.sync_copy(acc_sh.at[pl.ds(s * RPT, RPT)],
                        out_hbm.at[c, pl.ds(s * RPT, RPT)])

    return prop


_propagate = {D: _make_propagate(D) for D in (H1, H2, H3)}

_deg_mesh = plsc.VectorSubcoreMesh(core_axis_name="c", subcore_axis_name="s")


@functools.partial(
    pl.kernel,
    out_type=jax.ShapeDtypeStruct((NC, NP, DEGW), jnp.float32),
    mesh=_deg_mesh,
    scratch_types=[
        pltpu.VMEM((DEG_NCH, DEG_K), jnp.int32),
        pltpu.VMEM((DEG_K, DEGW), jnp.float32),
        pltpu.VMEM_SHARED((NP, DEGW), jnp.float32),
        pltpu.SemaphoreType.DMA,
    ],
    compiler_params=pltpu.CompilerParams(use_tc_tiling_on_sc=False),
)
def _deg_kernel(dst_hbm, ones_hbm, zeros_hbm, out_hbm,
                dst_v, ones_v, acc_sh, sem):
    c = lax.axis_index("c")
    s = lax.axis_index("s")
    wid = s * NC + c
    pltpu.sync_copy(dst_hbm.at[wid], dst_v)
    pltpu.sync_copy(ones_hbm, ones_v)
    pltpu.sync_copy(zeros_hbm, acc_sh.at[pl.ds(s * RPT, RPT)])
    plsc.subcore_barrier()

    def body(j, carry):
        pltpu.sync_copy(ones_v, acc_sh.at[dst_v.at[j]], add=True)
        return carry

    lax.fori_loop(0, DEG_NCH, body, 0)
    plsc.subcore_barrier()
    pltpu.sync_copy(acc_sh.at[pl.ds(s * RPT, RPT)],
                    out_hbm.at[c, pl.ds(s * RPT, RPT)])


_NB = N // _BR


def _mm1_body(x_ref, w_ref, dp_ref, y_ref, dinv_ref):
    dinv = lax.rsqrt(dp_ref[0, :, 0:1] + dp_ref[1, :, 0:1] + 1.0)
    dinv_ref[...] = dinv
    y_ref[...] = jnp.dot(x_ref[...], w_ref[...],
                         preferred_element_type=jnp.float32) * dinv


def _mm1(x, w, degp):
    din, dout = w.shape
    return pl.pallas_call(
        _mm1_body,
        grid=(_NB,),
        in_specs=[
            pl.BlockSpec((_BR, din), lambda i: (i, 0)),
            pl.BlockSpec((din, dout), lambda i: (0, 0)),
            pl.BlockSpec((NC, _BR, DEGW), lambda i: (0, i, 0)),
        ],
        out_specs=[
            pl.BlockSpec((_BR, dout), lambda i: (i, 0)),
            pl.BlockSpec((_BR, 1), lambda i: (i, 0)),
        ],
        out_shape=[
            jax.ShapeDtypeStruct((N, dout), jnp.float32),
            jax.ShapeDtypeStruct((N, 1), jnp.float32),
        ],
    )(x, w, degp)


def _comb_bn_mm_body(p_ref, y_ref, dinva_ref, g_ref, be_ref, w_ref, b_ref,
                     dinvb_ref, o_ref, h_buf, s_ref):
    i = pl.program_id(0)

    @pl.when(i < _NB)
    def _phase0():
        h = (p_ref[0] + p_ref[1] + y_ref[...]) * dinva_ref[...]
        start = pl.multiple_of(i * _BR, _BR)
        h_buf[pl.ds(start, _BR), :] = h

        @pl.when(i == 0)
        def _init():
            s_ref[...] = jnp.zeros_like(s_ref)

        s0 = jnp.sum(h, axis=0, keepdims=True)
        s1 = jnp.sum(h * h, axis=0, keepdims=True)
        s_ref[...] += jnp.concatenate([s0, s1], axis=0)

    @pl.when(i >= _NB)
    def _phase1():
        mean = s_ref[0:1, :] * (1.0 / N)
        var = s_ref[1:2, :] * (1.0 / N) - mean * mean
        scale = g_ref[...] * lax.rsqrt(var + _BN_EPS)
        shift = be_ref[...] - mean * scale
        start = pl.multiple_of((i - _NB) * _BR, _BR)
        h = jnp.maximum(h_buf[pl.ds(start, _BR), :] * scale + shift, 0.0)
        o_ref[...] = (jnp.dot(h, w_ref[...],
                              preferred_element_type=jnp.float32)
                      + b_ref[...]) * dinvb_ref[...]


def _comb_bn_mm(p, y, dinv, g, be, w, b, dinv2):
    din, dout = w.shape
    blk = lambda i: jnp.minimum(i, _NB - 1)
    out_blk = lambda i: jnp.maximum(i - _NB, 0)
    return pl.pallas_call(
        _comb_bn_mm_body,
        grid=(2 * _NB,),
        in_specs=[
            pl.BlockSpec((NC, _BR, din), lambda i: (0, blk(i), 0)),
            pl.BlockSpec((_BR, din), lambda i: (blk(i), 0)),
            pl.BlockSpec((_BR, 1), lambda i: (blk(i), 0)),
            pl.BlockSpec((1, din), lambda i: (0, 0)),
            pl.BlockSpec((1, din), lambda i: (0, 0)),
            pl.BlockSpec((din, dout), lambda i: (0, 0)),
            pl.BlockSpec((1, dout), lambda i: (0, 0)),
            pl.BlockSpec((_BR, 1), lambda i: (out_blk(i), 0)),
        ],
        out_specs=pl.BlockSpec((_BR, dout), lambda i: (out_blk(i), 0)),
        out_shape=jax.ShapeDtypeStruct((N, dout), jnp.float32),
        scratch_shapes=[
            pltpu.VMEM((N, din), jnp.float32),
            pltpu.VMEM((2, din), jnp.float32),
        ],
    )(p, y, dinv, g, be, w, b, dinv2)


def kernel(x, edge_index, W1, b1, g1, be1, W2, b2, g2, be2, W3, b3, g3, be3,
           Wf, bf):
    def pad_edges(e_row, k, nbuf, is_dst):
        epwp, nchp = _epwp(k, nbuf)
        pad = epwp - EPW
        base = e_row.reshape(NW, EPW).astype(jnp.int32)
        if pad:
            if is_dst:
                padv = jnp.broadcast_to(
                    (N + jnp.arange(pad, dtype=jnp.int32))[None, :], (NW, pad))
            else:
                padv = jnp.zeros((NW, pad), jnp.int32)
            base = jnp.concatenate([base, padv], axis=1)
        return base.reshape(NW, nchp, k)

    src1 = pad_edges(edge_index[0], _KCFG[H1], _NBUFCFG[H1], False)
    dst1 = pad_edges(edge_index[1], _KCFG[H1], _NBUFCFG[H1], True)
    src2 = pad_edges(edge_index[0], _KCFG[H2], _NBUFCFG[H2], False)
    dst2 = pad_edges(edge_index[1], _KCFG[H2], _NBUFCFG[H2], True)
    dst_deg = edge_index[1].reshape(NW, DEG_NCH, DEG_K).astype(jnp.int32)

    ones_deg = jnp.ones((DEG_K, DEGW), jnp.float32)
    zeros_deg = jnp.zeros((RPT, DEGW), jnp.float32)
    degp = _deg_kernel(dst_deg, ones_deg, zeros_deg)

    g1r, be1r = g1.reshape(1, H1), be1.reshape(1, H1)
    g2r, be2r = g2.reshape(1, H2), be2.reshape(1, H2)
    g3r, be3r = g3.reshape(1, H3), be3.reshape(1, H3)
    wf_pad = jnp.zeros((H3, 128), jnp.float32).at[:, 0:1].set(Wf)
    bf_pad = jnp.zeros((1, 128), jnp.float32).at[0, 0].set(bf[0])
    zeros2 = jnp.zeros((1, H2), jnp.float32)
    zeros3 = jnp.zeros((1, H3), jnp.float32)
    ones_n = jnp.ones((N, 1), jnp.float32)

    y1, dinv = _mm1(x, W1, degp)
    p1 = _propagate[H1](src1, dst1, y1, jnp.zeros((RPT, H1), jnp.float32))
    y2 = _comb_bn_mm(p1, y1, dinv, g1r, be1r, W2, zeros2, dinv)
    p2 = _propagate[H2](src2, dst2, y2, jnp.zeros((RPT, H2), jnp.float32))
    y3 = _comb_bn_mm(p2, y2, dinv, g2r, be2r, W3, zeros3, dinv)
    p3 = _propagate[H3](src2, dst2, y3, jnp.zeros((RPT, H3), jnp.float32))
    out = _comb_bn_mm(p3, y3, dinv, g3r, be3r, wf_pad, bf_pad, ones_n)
    return out[:, 0:1]


# speedup vs baseline: 1.2646x; 1.2646x over previous
import functools

import jax
import jax.numpy as jnp
from jax import lax
from jax.experimental import pallas as pl
from jax.experimental.pallas import tpu as pltpu
from jax.experimental.pallas import tpu_sc as plsc

N = 10000
E = 320000
D_IN = 128
H1, H2, H3 = 128, 64, 32

NC = 2
NS = 16
NW = NC * NS
EPW = E // NW
NP = 10240
RPT = NP // NS
DEG_K = 80
DEG_NCH = EPW // DEG_K
DEGW = 16

_BN_EPS = 1e-5
_BR = 5000


_KCFG = {H1: 80, H2: 80, H3: 80}
_NBUFCFG = {H1: 2, H2: 3, H3: 3}


def _epwp(k, nbuf):
    n = -(-EPW // k)
    while n % nbuf != nbuf - 1:
        n += 1
    return n * k, n


def _make_propagate(D):
    k = _KCFG[D]
    nbuf = _NBUFCFG[D]
    epwp, nchp = _epwp(k, nbuf)
    assert nchp % nbuf == nbuf - 1
    mesh = plsc.VectorSubcoreMesh(core_axis_name="c", subcore_axis_name="s")

    @functools.partial(
        pl.kernel,
        out_type=jax.ShapeDtypeStruct((NC, NP, D), jnp.float32),
        mesh=mesh,
        scratch_types=(
            [pltpu.VMEM((nchp, k), jnp.int32),
             pltpu.VMEM((nchp, k), jnp.int32)]
            + [pltpu.VMEM((k, D), jnp.float32) for _ in range(nbuf)]
            + [pltpu.VMEM_SHARED((NP, D), jnp.float32)]
            + [pltpu.SemaphoreType.DMA for _ in range(nbuf)]
        ),
        compiler_params=pltpu.CompilerParams(use_tc_tiling_on_sc=False),
    )
    def prop(src_hbm, dst_hbm, y_hbm, out_hbm,
             src_v, dst_v, *bufs):
        rows = list(bufs[0:nbuf])
        acc_sh = bufs[nbuf]
        sems = list(bufs[nbuf + 1:2 * nbuf + 1])
        c = lax.axis_index("c")
        s = lax.axis_index("s")
        wid = s * NC + c
        pltpu.sync_copy(src_hbm.at[wid], src_v)
        pltpu.sync_copy(dst_hbm.at[wid], dst_v)

        z16 = jnp.zeros((16,), jnp.float32)

        def zrow(r, carry):
            for cc in range(D // 16):
                rows[0][r, pl.ds(cc * 16, 16)] = z16
            return carry

        lax.fori_loop(0, k, zrow, 0)
        for t in range(RPT // k):
            pltpu.sync_copy(rows[0], acc_sh.at[pl.ds(s * RPT + t * k, k)])
        plsc.subcore_barrier()

        for b in range(nbuf - 1):
            pltpu.async_copy(y_hbm.at[src_v.at[b]], rows[b], sems[b])

        def body(i, carry):
            for b in range(nbuf):
                j = nbuf * i + b
                kn = (b + nbuf - 1) % nbuf
                pltpu.async_copy(y_hbm.at[src_v.at[j + nbuf - 1]],
                                 rows[kn], sems[kn])
                pltpu.make_async_copy(y_hbm.at[src_v.at[j]], rows[b],
                                      sems[b]).wait()
                pltpu.sync_copy(rows[b], acc_sh.at[dst_v.at[j]], add=True)
            return carry

        lax.fori_loop(0, (nchp - (nbuf - 1)) // nbuf, body, 0)
        for t in range(nbuf - 1):
            j = nchp - (nbuf - 1) + t
            b = j % nbuf
            pltpu.make_async_copy(y_hbm.at[src_v.at[j]], rows[b],
                                  sems[b]).wait()
            pltpu.sync_copy(rows[b], acc_sh.at[dst_v.at[j]], add=True)
        plsc.subcore_barrier()
        pltpu.sync_copy(acc_sh.at[pl.ds(s * RPT, RPT)],
                        out_hbm.at[c, pl.ds(s * RPT, RPT)])

    return prop


_propagate = {D: _make_propagate(D) for D in (H1, H2, H3)}

_deg_mesh = plsc.VectorSubcoreMesh(core_axis_name="c", subcore_axis_name="s")


@functools.partial(
    pl.kernel,
    out_type=jax.ShapeDtypeStruct((NC, NP, DEGW), jnp.float32),
    mesh=_deg_mesh,
    scratch_types=[
        pltpu.VMEM((DEG_NCH, DEG_K), jnp.int32),
        pltpu.VMEM((DEG_K, DEGW), jnp.float32),
        pltpu.VMEM((DEG_K, DEGW), jnp.float32),
        pltpu.VMEM_SHARED((NP, DEGW), jnp.float32),
        pltpu.SemaphoreType.DMA,
    ],
    compiler_params=pltpu.CompilerParams(use_tc_tiling_on_sc=False),
)
def _deg_kernel(dst_hbm, out_hbm, dst_v, ones_v, zero_v, acc_sh, sem):
    c = lax.axis_index("c")
    s = lax.axis_index("s")
    wid = s * NC + c
    pltpu.sync_copy(dst_hbm.at[wid], dst_v)

    one16 = jnp.ones((16,), jnp.float32)
    z16 = jnp.zeros((16,), jnp.float32)

    def initrow(r, carry):
        ones_v[r, pl.ds(0, DEGW)] = one16
        zero_v[r, pl.ds(0, DEGW)] = z16
        return carry

    lax.fori_loop(0, DEG_K, initrow, 0)
    for t in range(RPT // DEG_K):
        pltpu.sync_copy(zero_v, acc_sh.at[pl.ds(s * RPT + t * DEG_K, DEG_K)])
    plsc.subcore_barrier()

    def fire(j, carry):
        pltpu.async_copy(ones_v, acc_sh.at[dst_v.at[j]], sem, add=True)
        return carry

    lax.fori_loop(0, DEG_NCH, fire, 0)

    def drain(j, carry):
        pltpu.make_async_copy(ones_v, acc_sh.at[dst_v.at[0]], sem).wait()
        return carry

    lax.fori_loop(0, DEG_NCH, drain, 0)
    plsc.subcore_barrier()
    pltpu.sync_copy(acc_sh.at[pl.ds(s * RPT, RPT)],
                    out_hbm.at[c, pl.ds(s * RPT, RPT)])


_NB = N // _BR


def _mm1_body(x_ref, w_ref, dp_ref, y_ref, dinv_ref):
    dinv = lax.rsqrt(dp_ref[0, :, 0:1] + dp_ref[1, :, 0:1] + 1.0)
    dinv_ref[...] = dinv
    y_ref[...] = jnp.dot(x_ref[...], w_ref[...],
                         preferred_element_type=jnp.float32) * dinv


def _mm1(x, w, degp):
    din, dout = w.shape
    return pl.pallas_call(
        _mm1_body,
        grid=(_NB,),
        in_specs=[
            pl.BlockSpec((_BR, din), lambda i: (i, 0)),
            pl.BlockSpec((din, dout), lambda i: (0, 0)),
            pl.BlockSpec((NC, _BR, DEGW), lambda i: (0, i, 0)),
        ],
        out_specs=[
            pl.BlockSpec((_BR, dout), lambda i: (i, 0)),
            pl.BlockSpec((_BR, 1), lambda i: (i, 0)),
        ],
        out_shape=[
            jax.ShapeDtypeStruct((N, dout), jnp.float32),
            jax.ShapeDtypeStruct((N, 1), jnp.float32),
        ],
    )(x, w, degp)


def _comb_bn_mm_body(p_ref, y_ref, dinva_ref, g_ref, be_ref, w_ref, b_ref,
                     dinvb_ref, o_ref, h_buf, s_ref):
    i = pl.program_id(0)

    @pl.when(i < _NB)
    def _phase0():
        h = (p_ref[0] + p_ref[1] + y_ref[...]) * dinva_ref[...]
        start = pl.multiple_of(i * _BR, _BR)
        h_buf[pl.ds(start, _BR), :] = h

        @pl.when(i == 0)
        def _init():
            s_ref[...] = jnp.zeros_like(s_ref)

        s0 = jnp.sum(h, axis=0, keepdims=True)
        s1 = jnp.sum(h * h, axis=0, keepdims=True)
        s_ref[...] += jnp.concatenate([s0, s1], axis=0)

    @pl.when(i >= _NB)
    def _phase1():
        mean = s_ref[0:1, :] * (1.0 / N)
        var = s_ref[1:2, :] * (1.0 / N) - mean * mean
        scale = g_ref[...] * lax.rsqrt(var + _BN_EPS)
        shift = be_ref[...] - mean * scale
        start = pl.multiple_of((i - _NB) * _BR, _BR)
        h = jnp.maximum(h_buf[pl.ds(start, _BR), :] * scale + shift, 0.0)
        o_ref[...] = (jnp.dot(h, w_ref[...],
                              preferred_element_type=jnp.float32)
                      + b_ref[...]) * dinvb_ref[...]


def _comb_bn_mm(p, y, dinv, g, be, w, b, dinv2):
    din, dout = w.shape
    blk = lambda i: jnp.minimum(i, _NB - 1)
    out_blk = lambda i: jnp.maximum(i - _NB, 0)
    return pl.pallas_call(
        _comb_bn_mm_body,
        grid=(2 * _NB,),
        in_specs=[
            pl.BlockSpec((NC, _BR, din), lambda i: (0, blk(i), 0)),
            pl.BlockSpec((_BR, din), lambda i: (blk(i), 0)),
            pl.BlockSpec((_BR, 1), lambda i: (blk(i), 0)),
            pl.BlockSpec((1, din), lambda i: (0, 0)),
            pl.BlockSpec((1, din), lambda i: (0, 0)),
            pl.BlockSpec((din, dout), lambda i: (0, 0)),
            pl.BlockSpec((1, dout), lambda i: (0, 0)),
            pl.BlockSpec((_BR, 1), lambda i: (out_blk(i), 0)),
        ],
        out_specs=pl.BlockSpec((_BR, dout), lambda i: (out_blk(i), 0)),
        out_shape=jax.ShapeDtypeStruct((N, dout), jnp.float32),
        scratch_shapes=[
            pltpu.VMEM((N, din), jnp.float32),
            pltpu.VMEM((2, din), jnp.float32),
        ],
    )(p, y, dinv, g, be, w, b, dinv2)


def kernel(x, edge_index, W1, b1, g1, be1, W2, b2, g2, be2, W3, b3, g3, be3,
           Wf, bf):
    def pad_edges(e_row, k, nbuf, is_dst):
        epwp, nchp = _epwp(k, nbuf)
        pad = epwp - EPW
        base = e_row.reshape(NW, EPW).astype(jnp.int32)
        if pad:
            if is_dst:
                padv = jnp.broadcast_to(
                    (N + jnp.arange(pad, dtype=jnp.int32))[None, :], (NW, pad))
            else:
                padv = jnp.zeros((NW, pad), jnp.int32)
            base = jnp.concatenate([base, padv], axis=1)
        return base.reshape(NW, nchp, k)

    src1 = pad_edges(edge_index[0], _KCFG[H1], _NBUFCFG[H1], False)
    dst1 = pad_edges(edge_index[1], _KCFG[H1], _NBUFCFG[H1], True)
    src2 = pad_edges(edge_index[0], _KCFG[H2], _NBUFCFG[H2], False)
    dst2 = pad_edges(edge_index[1], _KCFG[H2], _NBUFCFG[H2], True)
    dst_deg = edge_index[1].reshape(NW, DEG_NCH, DEG_K).astype(jnp.int32)

    degp = _deg_kernel(dst_deg)

    g1r, be1r = g1.reshape(1, H1), be1.reshape(1, H1)
    g2r, be2r = g2.reshape(1, H2), be2.reshape(1, H2)
    g3r, be3r = g3.reshape(1, H3), be3.reshape(1, H3)
    wf_pad = jnp.zeros((H3, 128), jnp.float32).at[:, 0:1].set(Wf)
    bf_pad = jnp.zeros((1, 128), jnp.float32).at[0, 0].set(bf[0])
    zeros2 = jnp.zeros((1, H2), jnp.float32)
    zeros3 = jnp.zeros((1, H3), jnp.float32)
    ones_n = jnp.ones((N, 1), jnp.float32)

    y1, dinv = _mm1(x, W1, degp)
    p1 = _propagate[H1](src1, dst1, y1)
    y2 = _comb_bn_mm(p1, y1, dinv, g1r, be1r, W2, zeros2, dinv)
    p2 = _propagate[H2](src2, dst2, y2)
    y3 = _comb_bn_mm(p2, y2, dinv, g2r, be2r, W3, zeros3, dinv)
    p3 = _propagate[H3](src2, dst2, y3)
    out = _comb_bn_mm(p3, y3, dinv, g3r, be3r, wf_pad, bf_pad, ones_n)
    return out[:, 0:1]
